# baseline (speedup 1.0000x reference)
import functools
import math

import jax
import jax.numpy as jnp
from jax import lax
from jax.experimental import pallas as pl
from jax.experimental.pallas import tpu as pltpu

_SCALE = 64.0
_MARGIN = 0.5
_B = 1024
_C = 100000

_CH = 4
_NCH = _B // _CH
_NBUF = 8


def _compute_chunk(lab8, x, *, cos_m, sin_m):
    cols = lax.broadcasted_iota(jnp.int32, x.shape, 1)
    xc = jnp.clip(x, -1.0, 1.0)
    adj = xc * cos_m - jnp.sqrt(jnp.maximum(1.0 - xc * xc, 0.0)) * sin_m
    return jnp.where(cols == lab8, adj, xc) * _SCALE


def _ring_body(lab_ref, x_hbm, o_hbm, inb, outb, insem, outsem, *, cos_m, sin_m):
    def in_copy(ch, b):
        return pltpu.make_async_copy(
            x_hbm.at[pl.ds(ch * _CH, _CH)], inb.at[b], insem.at[b]
        )

    def out_copy(ch, b):
        return pltpu.make_async_copy(
            outb.at[b], o_hbm.at[pl.ds(ch * _CH, _CH)], outsem.at[b]
        )

    for b in range(_NBUF):
        in_copy(b, b).start()

    def group(g, carry):
        for b in range(_NBUF):
            ch = g * _NBUF + b
            in_copy(ch, b).wait()

            @pl.when(g > 0)
            def _():
                out_copy(ch - _NBUF, b).wait()

            lab8 = lab_ref[pl.ds(ch * _CH, _CH), :]
            outb[b] = _compute_chunk(lab8, inb[b], cos_m=cos_m, sin_m=sin_m)
            out_copy(ch, b).start()

            @pl.when(ch + _NBUF < _NCH)
            def _():
                in_copy(ch + _NBUF, b).start()

        return carry

    lax.fori_loop(0, _NCH // _NBUF, group, 0)
    for b in range(_NBUF):
        out_copy(_NCH - _NBUF + b, b).wait()


def kernel(logits, labels):
    body = functools.partial(
        _ring_body, cos_m=math.cos(_MARGIN), sin_m=math.sin(_MARGIN)
    )
    return pl.pallas_call(
        body,
        in_specs=[
            pl.BlockSpec(memory_space=pltpu.VMEM),
            pl.BlockSpec(memory_space=pl.ANY),
        ],
        out_specs=pl.BlockSpec(memory_space=pl.ANY),
        out_shape=jax.ShapeDtypeStruct((_B, _C), jnp.float32),
        scratch_shapes=[
            pltpu.VMEM((_NBUF, _CH, _C), jnp.float32),
            pltpu.VMEM((_NBUF, _CH, _C), jnp.float32),
            pltpu.SemaphoreType.DMA((_NBUF,)),
            pltpu.SemaphoreType.DMA((_NBUF,)),
        ],
    )(jnp.reshape(labels, (_B, 1)), logits)


# speedup vs baseline: 1.1172x; 1.1172x over previous
import functools
import math

import jax
import jax.numpy as jnp
from jax import lax
from jax.experimental import pallas as pl
from jax.experimental.pallas import tpu as pltpu

_SCALE = 64.0
_MARGIN = 0.5
_B = 1024
_C = 100000

_CH = 8
_NCH = _B // _CH
_NBUF = 4


def _compute_chunk(lab8, x, *, cos_m, sin_m):
    cols = lax.broadcasted_iota(jnp.int32, x.shape, 1)
    xc = jnp.clip(x, -1.0, 1.0)
    adj = xc * cos_m - jnp.sqrt(jnp.maximum(1.0 - xc * xc, 0.0)) * sin_m
    return jnp.where(cols == lab8, adj, xc) * _SCALE


def _ring_body(lab_ref, x_hbm, o_hbm, inb, outb, insem, outsem, *, cos_m, sin_m):
    def in_copy(ch, b):
        return pltpu.make_async_copy(
            x_hbm.at[pl.ds(ch * _CH, _CH)], inb.at[b], insem.at[b]
        )

    def out_copy(ch, b):
        return pltpu.make_async_copy(
            outb.at[b], o_hbm.at[pl.ds(ch * _CH, _CH)], outsem.at[b]
        )

    for b in range(_NBUF):
        in_copy(b, b).start()

    def group(g, carry):
        for b in range(_NBUF):
            ch = g * _NBUF + b
            in_copy(ch, b).wait()

            @pl.when(g > 0)
            def _():
                out_copy(ch - _NBUF, b).wait()

            lab8 = lab_ref[pl.ds(ch * _CH, _CH), :]
            outb[b] = _compute_chunk(lab8, inb[b], cos_m=cos_m, sin_m=sin_m)
            out_copy(ch, b).start()

            @pl.when(ch + _NBUF < _NCH)
            def _():
                in_copy(ch + _NBUF, b).start()

        return carry

    lax.fori_loop(0, _NCH // _NBUF, group, 0)
    for b in range(_NBUF):
        out_copy(_NCH - _NBUF + b, b).wait()


def kernel(logits, labels):
    body = functools.partial(
        _ring_body, cos_m=math.cos(_MARGIN), sin_m=math.sin(_MARGIN)
    )
    return pl.pallas_call(
        body,
        in_specs=[
            pl.BlockSpec(memory_space=pltpu.VMEM),
            pl.BlockSpec(memory_space=pl.ANY),
        ],
        out_specs=pl.BlockSpec(memory_space=pl.ANY),
        out_shape=jax.ShapeDtypeStruct((_B, _C), jnp.float32),
        scratch_shapes=[
            pltpu.VMEM((_NBUF, _CH, _C), jnp.float32),
            pltpu.VMEM((_NBUF, _CH, _C), jnp.float32),
            pltpu.SemaphoreType.DMA((_NBUF,)),
            pltpu.SemaphoreType.DMA((_NBUF,)),
        ],
    )(jnp.reshape(labels, (_B, 1)), logits)
